# baseline (speedup 1.0000x reference)
import functools

import jax
import jax.numpy as jnp
from jax import lax
from jax.experimental import pallas as pl
from jax.experimental.pallas import tpu as pltpu
from jax.experimental.pallas import tpu_sc as plsc

_VOCAB = 1_000_000
_D = 64
_B = 16384

_NC = 2
_NS = 16
_NW = _NC * _NS
_BPW = _B // _NW
_CHUNK = 128
_NCHUNK = _BPW // _CHUNK


def _allreduce(v, op):
    idx = lax.iota(jnp.int32, 16)
    for shift in (8, 4, 2, 1):
        perm = idx ^ shift
        v = op(v, v.at[perm].get(mode="promise_in_bounds"))
    return v


def _softmax_rows(rows_v, n_rows):
    def row_body(r, carry):
        v0 = rows_v[r, pl.ds(0, 16)]
        v1 = rows_v[r, pl.ds(16, 16)]
        v2 = rows_v[r, pl.ds(32, 16)]
        v3 = rows_v[r, pl.ds(48, 16)]
        m = _allreduce(
            jnp.maximum(jnp.maximum(v0, v1), jnp.maximum(v2, v3)), jnp.maximum
        )
        e0 = jnp.exp(v0 - m)
        e1 = jnp.exp(v1 - m)
        e2 = jnp.exp(v2 - m)
        e3 = jnp.exp(v3 - m)
        inv = 1.0 / _allreduce((e0 + e1) + (e2 + e3), jnp.add)
        rows_v[r, pl.ds(0, 16)] = e0 * inv
        rows_v[r, pl.ds(16, 16)] = e1 * inv
        rows_v[r, pl.ds(32, 16)] = e2 * inv
        rows_v[r, pl.ds(48, 16)] = e3 * inv
        return carry

    lax.fori_loop(0, n_rows, row_body, None)


@functools.partial(
    pl.kernel,
    out_type=jax.ShapeDtypeStruct((_B, _D), jnp.float32),
    mesh=plsc.VectorSubcoreMesh(core_axis_name="c", subcore_axis_name="s"),
    scratch_types=[
        pltpu.VMEM((_NCHUNK, _CHUNK), jnp.int32),
        pltpu.VMEM((_BPW, _D), jnp.float32),
        pltpu.SemaphoreType.DMA,
    ],
    compiler_params=pltpu.CompilerParams(use_tc_tiling_on_sc=False),
)
def _decoder_kernel(x_hbm, table_hbm, out_hbm, idx_v, rows_v, sem):
    wid = lax.axis_index("s") * _NC + lax.axis_index("c")
    base = wid * _BPW

    for j in range(_NCHUNK):
        pltpu.sync_copy(x_hbm.at[pl.ds(base + j * _CHUNK, _CHUNK)], idx_v.at[j])

    copies = []
    for j in range(_NCHUNK):
        copies.append(
            pltpu.async_copy(
                table_hbm.at[idx_v.at[j]],
                rows_v.at[pl.ds(j * _CHUNK, _CHUNK)],
                sem,
            )
        )
    for c in copies:
        c.wait()

    _softmax_rows(rows_v, _BPW)

    pltpu.sync_copy(rows_v, out_hbm.at[pl.ds(base, _BPW)])


def kernel(x, table):
    return _decoder_kernel(x.astype(jnp.int32), table)
